# baseline (speedup 1.0000x reference)
import functools

import numpy as np
import jax
import jax.numpy as jnp
from jax import lax
from jax.experimental import pallas as pl
from jax.experimental.pallas import tpu as pltpu
from jax.experimental.pallas import tpu_sc as plsc

NUM_UNITS = 64
_LANES = 16
_NC = 2
_NS = 16
_NW = _NC * _NS

_R = 400
_G = 80
_PADW = 128


@functools.lru_cache(maxsize=None)
def _pos_enc(length: int, width: int):
    pe = np.array(
        [[pos / np.power(10000, 2 * i / width) for i in range(width)]
         for pos in range(length)],
        dtype=np.float32,
    )
    pe[:, 0::2] = np.sin(pe[:, 0::2])
    pe[:, 1::2] = np.cos(pe[:, 1::2])
    return jnp.asarray(pe)


@functools.lru_cache(maxsize=None)
def _make_detile(vocab: int):
    n_full = vocab // 128
    tail = vocab - n_full * 128
    per_w = (n_full + _NW - 1) // _NW
    mesh = plsc.VectorSubcoreMesh(core_axis_name="c", subcore_axis_name="s")

    @functools.partial(
        pl.kernel,
        out_type=jax.ShapeDtypeStruct((vocab // 2, 128), jnp.float32),
        mesh=mesh,
        scratch_types=[
            pltpu.VMEM((NUM_UNITS, 128), jnp.float32),
            pltpu.VMEM((NUM_UNITS, 128), jnp.float32),
            pltpu.VMEM((NUM_UNITS, NUM_UNITS), jnp.float32),
            pltpu.VMEM((tail // 2 if tail else 2, 128), jnp.float32),
        ],
        compiler_params=pltpu.CompilerParams(
            use_tc_tiling_on_sc=True, needs_layout_passes=False
        ),
    )
    def k(tbl_t, out_tbl, tile_v, out_v, tail_v, outt_v):
        wid = lax.axis_index("c") * _NS + lax.axis_index("s")
        iot = jnp.arange(_LANES, dtype=jnp.int32)

        def transpose_cols(src_v, dst_v, n_cols):
            def vloop(v, carry):
                half = v // 2
                coff = (v % 2) * NUM_UNITS
                for j in range(NUM_UNITS // _LANES):
                    vals = plsc.load_gather(
                        src_v, [j * _LANES + iot, jnp.full((_LANES,), v, jnp.int32)]
                    )
                    dst_v[half, pl.ds(coff + j * _LANES, _LANES)] = vals
                return carry

            lax.fori_loop(0, n_cols, vloop, 0, unroll=False)

        def slab(i, carry):
            v_blk = wid * per_w + i

            @pl.when(v_blk < n_full)
            def _():
                pltpu.sync_copy(tbl_t.at[:, pl.ds(v_blk * 128, 128)], tile_v)
                transpose_cols(tile_v, out_v, 128)
                pltpu.sync_copy(out_v, out_tbl.at[pl.ds(v_blk * 64, 64)])

            return carry

        lax.fori_loop(0, per_w, slab, 0, unroll=False)

        if tail:
            @pl.when(wid == _NW - 1)
            def _():
                pltpu.sync_copy(
                    tbl_t.at[:, pl.ds(n_full * 128, tail)],
                    tail_v.at[:, pl.ds(0, tail)],
                )
                transpose_cols(tail_v, outt_v, tail)
                pltpu.sync_copy(
                    outt_v, out_tbl.at[pl.ds(n_full * 64, tail // 2)]
                )

    return k


@functools.lru_cache(maxsize=None)
def _make_kernel(n_rows: int, seq_len: int):
    rows_per_w = n_rows // _NW
    n_chunks = rows_per_w // _R
    half = n_chunks // 2
    reps = _R // seq_len
    n_sub = _R // _G
    n_vec = NUM_UNITS // _LANES
    mesh = plsc.VectorSubcoreMesh(core_axis_name="c", subcore_axis_name="s")

    @functools.partial(
        pl.kernel,
        out_type=jax.ShapeDtypeStruct((n_rows, _PADW), jnp.float32),
        mesh=mesh,
        scratch_types=[
            pltpu.VMEM((_R,), jnp.int32),
            pltpu.VMEM((_R,), jnp.int32),
            pltpu.VMEM((_R, NUM_UNITS), jnp.float32),
            pltpu.VMEM((_R, NUM_UNITS), jnp.float32),
            pltpu.VMEM((seq_len, NUM_UNITS), jnp.float32),
            pltpu.SemaphoreType.DMA,
            pltpu.SemaphoreType.DMA,
            pltpu.SemaphoreType.DMA,
            pltpu.SemaphoreType.DMA,
        ],
        compiler_params=pltpu.CompilerParams(use_tc_tiling_on_sc=False),
    )
    def k(ids_hbm, table_hbm, pe_hbm, out_hbm,
          idx0, idx1, rows0, rows1, pe_v, sg0, sg1, sw0, sw1):
        wid = lax.axis_index("c") * _NS + lax.axis_index("s")
        base = wid * rows_per_w
        pltpu.sync_copy(pe_hbm, pe_v)

        def out_slice(cb):
            return out_hbm.at[pl.ds(cb, _R), pl.ds(0, NUM_UNITS)]

        def fire_gathers(idx_v, rows_v, sem):
            return [
                pltpu.async_copy(
                    table_hbm.at[idx_v.at[pl.ds(j * _G, _G)]],
                    rows_v.at[pl.ds(j * _G, _G)],
                    sem,
                )
                for j in range(n_sub)
            ]

        def wait_gathers(idx_v, rows_v, sem):
            for j in range(n_sub):
                pltpu.make_async_copy(
                    table_hbm.at[idx_v.at[pl.ds(j * _G, _G)]],
                    rows_v.at[pl.ds(j * _G, _G)],
                    sem,
                ).wait()

        def add_pe(rows_v):
            def body(l, carry):
                pvs = [pe_v[l, pl.ds(j * _LANES, _LANES)] for j in range(n_vec)]
                for rep in range(reps):
                    r = rep * seq_len + l
                    for j in range(n_vec):
                        s = pl.ds(j * _LANES, _LANES)
                        rows_v[r, s] = rows_v[r, s] + pvs[j]
                return carry

            lax.fori_loop(0, seq_len, body, 0, unroll=False)

        pltpu.sync_copy(ids_hbm.at[pl.ds(base, _R)], idx0)
        fire_gathers(idx0, rows0, sg0)

        def pair(i, carry):
            c = 2 * i
            cb0 = base + c * _R
            cb1 = cb0 + _R

            @pl.when(i > 0)
            def _():
                pltpu.make_async_copy(
                    rows1, out_slice(cb1 - 2 * _R), sw1
                ).wait()

            pltpu.sync_copy(ids_hbm.at[pl.ds(cb1, _R)], idx1)
            fire_gathers(idx1, rows1, sg1)

            wait_gathers(idx0, rows0, sg0)
            add_pe(rows0)
            w0 = pltpu.async_copy(rows0, out_slice(cb0), sw0)

            nb = base + jnp.minimum(c + 2, n_chunks - 1) * _R
            pltpu.sync_copy(ids_hbm.at[pl.ds(nb, _R)], idx0)
            w0.wait()
            fire_gathers(idx0, rows0, sg0)

            wait_gathers(idx1, rows1, sg1)
            add_pe(rows1)
            pltpu.async_copy(rows1, out_slice(cb1), sw1)
            return carry

        lax.fori_loop(0, half, pair, 0, unroll=False)

        wait_gathers(idx0, rows0, sg0)
        pltpu.make_async_copy(
            rows1, out_slice(base + (n_chunks - 1) * _R), sw1
        ).wait()

    return k


def kernel(ids, table):
    b, seq_len = ids.shape
    vocab = table.shape[0]
    ids32 = ids.reshape(-1).astype(jnp.int32)
    pe = _pos_enc(seq_len, NUM_UNITS)
    tbl_compact = _make_detile(vocab)(jnp.swapaxes(table, 0, 1))
    tbl_rows = tbl_compact.reshape(vocab, NUM_UNITS)
    out = _make_kernel(b * seq_len, seq_len)(ids32, tbl_rows, pe)
    return out.reshape(b, seq_len, _PADW)[:, :, :NUM_UNITS]


# speedup vs baseline: 1.1610x; 1.1610x over previous
import functools

import numpy as np
import jax
import jax.numpy as jnp
from jax import lax
from jax.experimental import pallas as pl
from jax.experimental.pallas import tpu as pltpu
from jax.experimental.pallas import tpu_sc as plsc

NUM_UNITS = 64
_LANES = 16
_NC = 2
_NS = 16
_NW = _NC * _NS

_R = 400
_G = 80
_PADW = 128


@functools.lru_cache(maxsize=None)
def _pos_enc(length: int, width: int):
    pe = np.array(
        [[pos / np.power(10000, 2 * i / width) for i in range(width)]
         for pos in range(length)],
        dtype=np.float32,
    )
    pe[:, 0::2] = np.sin(pe[:, 0::2])
    pe[:, 1::2] = np.cos(pe[:, 1::2])
    return jnp.asarray(pe)


@functools.lru_cache(maxsize=None)
def _make_detile(vocab: int):
    n_tc = vocab // 128
    tail = vocab - n_tc * 128
    gc = 2
    gv = gc * 128
    n_grp = n_tc // gc
    per_w = -(-n_grp // _NW)
    per_w += per_w % 2
    halfg = per_w // 2
    mesh = plsc.VectorSubcoreMesh(core_axis_name="c", subcore_axis_name="s")

    @functools.partial(
        pl.kernel,
        out_type=jax.ShapeDtypeStruct((vocab // 2, 128), jnp.float32),
        mesh=mesh,
        scratch_types=[
            pltpu.VMEM((2, 8, 8, gv), jnp.float32),
            pltpu.VMEM((2, gv // 2, 128), jnp.float32),
            pltpu.VMEM((8, 8, tail if tail else 8), jnp.float32),
            pltpu.VMEM((tail // 2 if tail else 4, 128), jnp.float32),
            pltpu.SemaphoreType.DMA,
            pltpu.SemaphoreType.DMA,
            pltpu.SemaphoreType.DMA,
            pltpu.SemaphoreType.DMA,
        ],
        compiler_params=pltpu.CompilerParams(
            use_tc_tiling_on_sc=True, needs_layout_passes=False
        ),
    )
    def k(tbl_t, out_tbl, stage, outb, tail_v, outt_v, sr0, sr1, sw0, sw1):
        wid = lax.axis_index("c") * _NS + lax.axis_index("s")
        iot = jnp.arange(_LANES, dtype=jnp.int32)
        i0s = [(iot + j * _LANES) // 8 for j in range(NUM_UNITS // _LANES)]
        i1s = [(iot + j * _LANES) % 8 for j in range(NUM_UNITS // _LANES)]

        def grp(i):
            return (wid * per_w + i) % n_grp

        def read_cps(g, buf, sem):
            v0 = g * gv
            return [
                pltpu.async_copy(
                    tbl_t.at[pl.ds(8 * ub, 8), pl.ds(v0, gv)],
                    stage.at[buf, ub],
                    sem,
                )
                for ub in range(8)
            ]

        def wait_reads(g, buf, sem):
            v0 = g * gv
            for ub in range(8):
                pltpu.make_async_copy(
                    tbl_t.at[pl.ds(8 * ub, 8), pl.ds(v0, gv)],
                    stage.at[buf, ub],
                    sem,
                ).wait()

        def transpose(sbuf, obuf, n_cols):
            def vloop(v, vv):
                half = v // 2
                coff = (v % 2) * NUM_UNITS
                for j in range(NUM_UNITS // _LANES):
                    vals = plsc.load_gather(sbuf, [i0s[j], i1s[j], vv])
                    obuf[half, pl.ds(coff + j * _LANES, _LANES)] = vals
                return vv + 1

            lax.fori_loop(0, n_cols, vloop, jnp.zeros((_LANES,), jnp.int32),
                          unroll=False)

        def wr(g, buf, sem):
            return pltpu.async_copy(
                outb.at[buf], out_tbl.at[pl.ds(g * (gv // 2), gv // 2)], sem
            )

        def wr_wait(g, buf, sem):
            pltpu.make_async_copy(
                outb.at[buf], out_tbl.at[pl.ds(g * (gv // 2), gv // 2)], sem
            ).wait()

        read_cps(grp(0), 0, sr0)

        def pair(i, carry):
            g0 = grp(2 * i)
            g1 = grp(2 * i + 1)
            read_cps(g1, 1, sr1)
            wait_reads(g0, 0, sr0)

            @pl.when(i > 0)
            def _():
                wr_wait(grp(2 * i - 2), 0, sw0)

            transpose(stage.at[0], outb.at[0], gv)
            wr(g0, 0, sw0)
            read_cps(grp(2 * i + 2), 0, sr0)
            wait_reads(g1, 1, sr1)

            @pl.when(i > 0)
            def _():
                wr_wait(grp(2 * i - 1), 1, sw1)

            transpose(stage.at[1], outb.at[1], gv)
            wr(g1, 1, sw1)
            return carry

        lax.fori_loop(0, halfg, pair, 0, unroll=False)

        wait_reads(grp(2 * halfg), 0, sr0)
        wr_wait(grp(2 * halfg - 2), 0, sw0)
        wr_wait(grp(2 * halfg - 1), 1, sw1)

        if tail:
            @pl.when(wid == _NW - 1)
            def _():
                for ub in range(8):
                    pltpu.sync_copy(
                        tbl_t.at[pl.ds(8 * ub, 8), pl.ds(n_tc * 128, tail)],
                        tail_v.at[ub],
                    )
                transpose(tail_v, outt_v, tail)
                pltpu.sync_copy(
                    outt_v, out_tbl.at[pl.ds(n_tc * 64, tail // 2)]
                )

    return k


@functools.lru_cache(maxsize=None)
def _make_kernel(n_rows: int, seq_len: int):
    rows_per_w = n_rows // _NW
    n_chunks = rows_per_w // _R
    half = n_chunks // 2
    reps = _R // seq_len
    n_sub = _R // _G
    n_vec = NUM_UNITS // _LANES
    mesh = plsc.VectorSubcoreMesh(core_axis_name="c", subcore_axis_name="s")

    @functools.partial(
        pl.kernel,
        out_type=jax.ShapeDtypeStruct((n_rows, _PADW), jnp.float32),
        mesh=mesh,
        scratch_types=[
            pltpu.VMEM((_R,), jnp.int32),
            pltpu.VMEM((_R,), jnp.int32),
            pltpu.VMEM((_R, NUM_UNITS), jnp.float32),
            pltpu.VMEM((_R, NUM_UNITS), jnp.float32),
            pltpu.VMEM((seq_len, NUM_UNITS), jnp.float32),
            pltpu.SemaphoreType.DMA,
            pltpu.SemaphoreType.DMA,
            pltpu.SemaphoreType.DMA,
            pltpu.SemaphoreType.DMA,
        ],
        compiler_params=pltpu.CompilerParams(use_tc_tiling_on_sc=False),
    )
    def k(ids_hbm, table_hbm, pe_hbm, out_hbm,
          idx0, idx1, rows0, rows1, pe_v, sg0, sg1, sw0, sw1):
        wid = lax.axis_index("c") * _NS + lax.axis_index("s")
        base = wid * rows_per_w
        pltpu.sync_copy(pe_hbm, pe_v)

        def out_slice(cb):
            return out_hbm.at[pl.ds(cb, _R), pl.ds(0, NUM_UNITS)]

        def fire_gathers(idx_v, rows_v, sem):
            return [
                pltpu.async_copy(
                    table_hbm.at[idx_v.at[pl.ds(j * _G, _G)]],
                    rows_v.at[pl.ds(j * _G, _G)],
                    sem,
                )
                for j in range(n_sub)
            ]

        def wait_gathers(idx_v, rows_v, sem):
            for j in range(n_sub):
                pltpu.make_async_copy(
                    table_hbm.at[idx_v.at[pl.ds(j * _G, _G)]],
                    rows_v.at[pl.ds(j * _G, _G)],
                    sem,
                ).wait()

        def add_pe(rows_v):
            def body(l, carry):
                pvs = [pe_v[l, pl.ds(j * _LANES, _LANES)] for j in range(n_vec)]
                for rep in range(reps):
                    r = rep * seq_len + l
                    for j in range(n_vec):
                        s = pl.ds(j * _LANES, _LANES)
                        rows_v[r, s] = rows_v[r, s] + pvs[j]
                return carry

            lax.fori_loop(0, seq_len, body, 0, unroll=False)

        pltpu.sync_copy(ids_hbm.at[pl.ds(base, _R)], idx0)
        fire_gathers(idx0, rows0, sg0)

        def pair(i, carry):
            c = 2 * i
            cb0 = base + c * _R
            cb1 = cb0 + _R

            @pl.when(i > 0)
            def _():
                pltpu.make_async_copy(
                    rows1, out_slice(cb1 - 2 * _R), sw1
                ).wait()

            pltpu.sync_copy(ids_hbm.at[pl.ds(cb1, _R)], idx1)
            fire_gathers(idx1, rows1, sg1)

            wait_gathers(idx0, rows0, sg0)
            add_pe(rows0)
            w0 = pltpu.async_copy(rows0, out_slice(cb0), sw0)

            nb = base + jnp.minimum(c + 2, n_chunks - 1) * _R
            pltpu.sync_copy(ids_hbm.at[pl.ds(nb, _R)], idx0)
            w0.wait()
            fire_gathers(idx0, rows0, sg0)

            wait_gathers(idx1, rows1, sg1)
            add_pe(rows1)
            pltpu.async_copy(rows1, out_slice(cb1), sw1)
            return carry

        lax.fori_loop(0, half, pair, 0, unroll=False)

        wait_gathers(idx0, rows0, sg0)
        pltpu.make_async_copy(
            rows1, out_slice(base + (n_chunks - 1) * _R), sw1
        ).wait()

    return k


def kernel(ids, table):
    b, seq_len = ids.shape
    vocab = table.shape[0]
    ids32 = ids.reshape(-1).astype(jnp.int32)
    pe = _pos_enc(seq_len, NUM_UNITS)
    tbl_compact = _make_detile(vocab)(jnp.swapaxes(table, 0, 1))
    tbl_rows = tbl_compact.reshape(vocab, NUM_UNITS)
    out = _make_kernel(b * seq_len, seq_len)(ids32, tbl_rows, pe)
    return out.reshape(b, seq_len, _PADW)[:, :, :NUM_UNITS]


# speedup vs baseline: 1.1707x; 1.0084x over previous
import functools

import numpy as np
import jax
import jax.numpy as jnp
from jax import lax
from jax.experimental import pallas as pl
from jax.experimental.pallas import tpu as pltpu
from jax.experimental.pallas import tpu_sc as plsc

NUM_UNITS = 64
_LANES = 16
_NC = 2
_NS = 16
_NW = _NC * _NS

_R = 400
_G = 80
_PADW = 128


@functools.lru_cache(maxsize=None)
def _pos_enc(length: int, width: int):
    pe = np.array(
        [[pos / np.power(10000, 2 * i / width) for i in range(width)]
         for pos in range(length)],
        dtype=np.float32,
    )
    pe[:, 0::2] = np.sin(pe[:, 0::2])
    pe[:, 1::2] = np.cos(pe[:, 1::2])
    return jnp.asarray(pe)


@functools.lru_cache(maxsize=None)
def _make_detile(vocab: int):
    n_tc = vocab // 128
    tail = vocab - n_tc * 128
    gc = 2
    gv = gc * 128
    n_grp = n_tc // gc
    per_w = -(-n_grp // _NW)
    per_w += per_w % 2
    halfg = per_w // 2
    mesh = plsc.VectorSubcoreMesh(core_axis_name="c", subcore_axis_name="s")

    @functools.partial(
        pl.kernel,
        out_type=jax.ShapeDtypeStruct((vocab // 2, 128), jnp.float32),
        mesh=mesh,
        scratch_types=[
            pltpu.VMEM((2, 64, gv), jnp.float32),
            pltpu.VMEM((2, gv // 2, 128), jnp.float32),
            pltpu.VMEM((64, tail if tail else 8), jnp.float32),
            pltpu.VMEM((tail // 2 if tail else 4, 128), jnp.float32),
            pltpu.SemaphoreType.DMA,
            pltpu.SemaphoreType.DMA,
            pltpu.SemaphoreType.DMA,
            pltpu.SemaphoreType.DMA,
        ],
        compiler_params=pltpu.CompilerParams(
            use_tc_tiling_on_sc=True, needs_layout_passes=False
        ),
    )
    def k(tbl_t, out_tbl, stage, outb, tail_v, outt_v, sr0, sr1, sw0, sw1):
        wid = lax.axis_index("c") * _NS + lax.axis_index("s")
        iot = jnp.arange(_LANES, dtype=jnp.int32)
        u16s = [iot + j * _LANES for j in range(NUM_UNITS // _LANES)]

        def grp(i):
            return (wid * per_w + i) % n_grp

        def read_cps(g, buf, sem):
            v0 = g * gv
            return [
                pltpu.async_copy(
                    tbl_t.at[pl.ds(8 * ub, 8), pl.ds(v0, gv)],
                    stage.at[buf, pl.ds(8 * ub, 8)],
                    sem,
                )
                for ub in range(8)
            ]

        def wait_reads(g, buf, sem):
            v0 = g * gv
            for ub in range(8):
                pltpu.make_async_copy(
                    tbl_t.at[pl.ds(8 * ub, 8), pl.ds(v0, gv)],
                    stage.at[buf, pl.ds(8 * ub, 8)],
                    sem,
                ).wait()

        def transpose(sbuf, obuf, n_cols):
            def hloop(h, vv):
                for half_col in range(2):
                    coff = half_col * NUM_UNITS
                    for j in range(NUM_UNITS // _LANES):
                        vals = plsc.load_gather(sbuf, [u16s[j], vv])
                        obuf[h, pl.ds(coff + j * _LANES, _LANES)] = vals
                    vv = vv + 1
                return vv

            lax.fori_loop(0, n_cols // 2, hloop,
                          jnp.zeros((_LANES,), jnp.int32), unroll=2)

        def wr(g, buf, sem):
            return pltpu.async_copy(
                outb.at[buf], out_tbl.at[pl.ds(g * (gv // 2), gv // 2)], sem
            )

        def wr_wait(g, buf, sem):
            pltpu.make_async_copy(
                outb.at[buf], out_tbl.at[pl.ds(g * (gv // 2), gv // 2)], sem
            ).wait()

        read_cps(grp(0), 0, sr0)

        def pair(i, carry):
            g0 = grp(2 * i)
            g1 = grp(2 * i + 1)
            read_cps(g1, 1, sr1)
            wait_reads(g0, 0, sr0)

            @pl.when(i > 0)
            def _():
                wr_wait(grp(2 * i - 2), 0, sw0)

            transpose(stage.at[0], outb.at[0], gv)
            wr(g0, 0, sw0)
            read_cps(grp(2 * i + 2), 0, sr0)
            wait_reads(g1, 1, sr1)

            @pl.when(i > 0)
            def _():
                wr_wait(grp(2 * i - 1), 1, sw1)

            transpose(stage.at[1], outb.at[1], gv)
            wr(g1, 1, sw1)
            return carry

        lax.fori_loop(0, halfg, pair, 0, unroll=False)

        wait_reads(grp(2 * halfg), 0, sr0)
        wr_wait(grp(2 * halfg - 2), 0, sw0)
        wr_wait(grp(2 * halfg - 1), 1, sw1)

        if tail:
            @pl.when(wid == _NW - 1)
            def _():
                for ub in range(8):
                    pltpu.sync_copy(
                        tbl_t.at[pl.ds(8 * ub, 8), pl.ds(n_tc * 128, tail)],
                        tail_v.at[pl.ds(8 * ub, 8)],
                    )
                transpose(tail_v, outt_v, tail)
                pltpu.sync_copy(
                    outt_v, out_tbl.at[pl.ds(n_tc * 64, tail // 2)]
                )

    return k


@functools.lru_cache(maxsize=None)
def _make_kernel(n_rows: int, seq_len: int):
    rows_per_w = n_rows // _NW
    n_chunks = rows_per_w // _R
    half = n_chunks // 2
    reps = _R // seq_len
    n_sub = _R // _G
    n_vec = NUM_UNITS // _LANES
    mesh = plsc.VectorSubcoreMesh(core_axis_name="c", subcore_axis_name="s")

    @functools.partial(
        pl.kernel,
        out_type=jax.ShapeDtypeStruct((n_rows, _PADW), jnp.float32),
        mesh=mesh,
        scratch_types=[
            pltpu.VMEM((_R,), jnp.int32),
            pltpu.VMEM((_R,), jnp.int32),
            pltpu.VMEM((_R, NUM_UNITS), jnp.float32),
            pltpu.VMEM((_R, NUM_UNITS), jnp.float32),
            pltpu.VMEM((seq_len, NUM_UNITS), jnp.float32),
            pltpu.SemaphoreType.DMA,
            pltpu.SemaphoreType.DMA,
            pltpu.SemaphoreType.DMA,
            pltpu.SemaphoreType.DMA,
        ],
        compiler_params=pltpu.CompilerParams(use_tc_tiling_on_sc=False),
    )
    def k(ids_hbm, table_hbm, pe_hbm, out_hbm,
          idx0, idx1, rows0, rows1, pe_v, sg0, sg1, sw0, sw1):
        wid = lax.axis_index("c") * _NS + lax.axis_index("s")
        base = wid * rows_per_w
        pltpu.sync_copy(pe_hbm, pe_v)

        def out_slice(cb):
            return out_hbm.at[pl.ds(cb, _R), pl.ds(0, NUM_UNITS)]

        def fire_gathers(idx_v, rows_v, sem):
            return [
                pltpu.async_copy(
                    table_hbm.at[idx_v.at[pl.ds(j * _G, _G)]],
                    rows_v.at[pl.ds(j * _G, _G)],
                    sem,
                )
                for j in range(n_sub)
            ]

        def wait_gathers(idx_v, rows_v, sem):
            for j in range(n_sub):
                pltpu.make_async_copy(
                    table_hbm.at[idx_v.at[pl.ds(j * _G, _G)]],
                    rows_v.at[pl.ds(j * _G, _G)],
                    sem,
                ).wait()

        def add_pe(rows_v):
            def body(l, carry):
                pvs = [pe_v[l, pl.ds(j * _LANES, _LANES)] for j in range(n_vec)]
                for rep in range(reps):
                    r = rep * seq_len + l
                    for j in range(n_vec):
                        s = pl.ds(j * _LANES, _LANES)
                        rows_v[r, s] = rows_v[r, s] + pvs[j]
                return carry

            lax.fori_loop(0, seq_len, body, 0, unroll=False)

        pltpu.sync_copy(ids_hbm.at[pl.ds(base, _R)], idx0)
        fire_gathers(idx0, rows0, sg0)

        def pair(i, carry):
            c = 2 * i
            cb0 = base + c * _R
            cb1 = cb0 + _R

            @pl.when(i > 0)
            def _():
                pltpu.make_async_copy(
                    rows1, out_slice(cb1 - 2 * _R), sw1
                ).wait()

            pltpu.sync_copy(ids_hbm.at[pl.ds(cb1, _R)], idx1)
            fire_gathers(idx1, rows1, sg1)

            wait_gathers(idx0, rows0, sg0)
            add_pe(rows0)
            w0 = pltpu.async_copy(rows0, out_slice(cb0), sw0)

            nb = base + jnp.minimum(c + 2, n_chunks - 1) * _R
            pltpu.sync_copy(ids_hbm.at[pl.ds(nb, _R)], idx0)
            w0.wait()
            fire_gathers(idx0, rows0, sg0)

            wait_gathers(idx1, rows1, sg1)
            add_pe(rows1)
            pltpu.async_copy(rows1, out_slice(cb1), sw1)
            return carry

        lax.fori_loop(0, half, pair, 0, unroll=False)

        wait_gathers(idx0, rows0, sg0)
        pltpu.make_async_copy(
            rows1, out_slice(base + (n_chunks - 1) * _R), sw1
        ).wait()

    return k


def kernel(ids, table):
    b, seq_len = ids.shape
    vocab = table.shape[0]
    ids32 = ids.reshape(-1).astype(jnp.int32)
    pe = _pos_enc(seq_len, NUM_UNITS)
    tbl_compact = _make_detile(vocab)(jnp.swapaxes(table, 0, 1))
    tbl_rows = tbl_compact.reshape(vocab, NUM_UNITS)
    out = _make_kernel(b * seq_len, seq_len)(ids32, tbl_rows, pe)
    return out.reshape(b, seq_len, _PADW)[:, :, :NUM_UNITS]


# speedup vs baseline: 2.1817x; 1.8636x over previous
import functools

import numpy as np
import jax
import jax.numpy as jnp
from jax import lax
from jax.experimental import pallas as pl
from jax.experimental.pallas import tpu as pltpu
from jax.experimental.pallas import tpu_sc as plsc

NUM_UNITS = 64
_LANES = 16
_NC = 2
_NS = 16
_NW = _NC * _NS

_R = 400
_G = 80
_PADW = 128


@functools.lru_cache(maxsize=None)
def _pos_enc(length: int, width: int):
    pe = np.array(
        [[pos / np.power(10000, 2 * i / width) for i in range(width)]
         for pos in range(length)],
        dtype=np.float32,
    )
    pe[:, 0::2] = np.sin(pe[:, 0::2])
    pe[:, 1::2] = np.cos(pe[:, 1::2])
    return jnp.asarray(pe)


@functools.lru_cache(maxsize=None)
def _make_kernel(n_rows: int, seq_len: int):
    rows_per_w = n_rows // _NW
    n_chunks = rows_per_w // _R
    half = n_chunks // 2
    reps = _R // seq_len
    n_sub = _R // _G
    n_vec = NUM_UNITS // _LANES
    mesh = plsc.VectorSubcoreMesh(core_axis_name="c", subcore_axis_name="s")

    @functools.partial(
        pl.kernel,
        out_type=jax.ShapeDtypeStruct((n_rows, _PADW), jnp.float32),
        mesh=mesh,
        scratch_types=[
            pltpu.VMEM((_R,), jnp.int32),
            pltpu.VMEM((_R,), jnp.int32),
            pltpu.VMEM((_R, _PADW), jnp.float32),
            pltpu.VMEM((_R, _PADW), jnp.float32),
            pltpu.VMEM((seq_len, NUM_UNITS), jnp.float32),
            pltpu.SemaphoreType.DMA,
            pltpu.SemaphoreType.DMA,
            pltpu.SemaphoreType.DMA,
            pltpu.SemaphoreType.DMA,
        ],
        compiler_params=pltpu.CompilerParams(use_tc_tiling_on_sc=False),
    )
    def k(ids_hbm, table_hbm, pe_hbm, out_hbm,
          idx0, idx1, rows0, rows1, pe_v, sg0, sg1, sw0, sw1):
        wid = lax.axis_index("c") * _NS + lax.axis_index("s")
        base = wid * rows_per_w
        pltpu.sync_copy(pe_hbm, pe_v)

        def out_slice(cb):
            return out_hbm.at[pl.ds(cb, _R), pl.ds(0, NUM_UNITS)]

        def rows_data(rows_v):
            return rows_v.at[:, pl.ds(0, NUM_UNITS)]

        def fire_gathers(idx_v, rows_v, sem):
            return [
                pltpu.async_copy(
                    table_hbm.at[idx_v.at[pl.ds(j * _G, _G)]],
                    rows_v.at[pl.ds(j * _G, _G)],
                    sem,
                )
                for j in range(n_sub)
            ]

        def wait_gathers(idx_v, rows_v, sem):
            for j in range(n_sub):
                pltpu.make_async_copy(
                    table_hbm.at[idx_v.at[pl.ds(j * _G, _G)]],
                    rows_v.at[pl.ds(j * _G, _G)],
                    sem,
                ).wait()

        def add_pe(rows_v):
            def body(l, carry):
                pvs = [pe_v[l, pl.ds(j * _LANES, _LANES)] for j in range(n_vec)]
                for rep in range(reps):
                    r = rep * seq_len + l
                    for j in range(n_vec):
                        s = pl.ds(j * _LANES, _LANES)
                        rows_v[r, s] = rows_v[r, s] + pvs[j]
                return carry

            lax.fori_loop(0, seq_len, body, 0, unroll=False)

        pltpu.sync_copy(ids_hbm.at[pl.ds(base, _R)], idx0)
        fire_gathers(idx0, rows0, sg0)

        def pair(i, carry):
            c = 2 * i
            cb0 = base + c * _R
            cb1 = cb0 + _R

            @pl.when(i > 0)
            def _():
                pltpu.make_async_copy(
                    rows_data(rows1), out_slice(cb1 - 2 * _R), sw1
                ).wait()

            pltpu.sync_copy(ids_hbm.at[pl.ds(cb1, _R)], idx1)
            fire_gathers(idx1, rows1, sg1)

            wait_gathers(idx0, rows0, sg0)
            add_pe(rows0)
            w0 = pltpu.async_copy(rows_data(rows0), out_slice(cb0), sw0)

            nb = base + jnp.minimum(c + 2, n_chunks - 1) * _R
            pltpu.sync_copy(ids_hbm.at[pl.ds(nb, _R)], idx0)
            w0.wait()
            fire_gathers(idx0, rows0, sg0)

            wait_gathers(idx1, rows1, sg1)
            add_pe(rows1)
            pltpu.async_copy(rows_data(rows1), out_slice(cb1), sw1)
            return carry

        lax.fori_loop(0, half, pair, 0, unroll=False)

        wait_gathers(idx0, rows0, sg0)
        pltpu.make_async_copy(
            rows_data(rows1), out_slice(base + (n_chunks - 1) * _R), sw1
        ).wait()

    return k


def kernel(ids, table):
    b, seq_len = ids.shape
    vocab = table.shape[0]
    ids32 = ids.reshape(-1).astype(jnp.int32)
    pe = _pos_enc(seq_len, NUM_UNITS)
    tblp = jnp.pad(table, ((0, 0), (0, _PADW - NUM_UNITS)))
    out = _make_kernel(b * seq_len, seq_len)(ids32, tblp, pe)
    return out.reshape(b, seq_len, _PADW)[:, :, :NUM_UNITS]
